# Initial kernel scaffold; baseline (speedup 1.0000x reference)
import jax
import jax.numpy as jnp
from jax.experimental import pallas as pl


def kernel(x, table):
    raise NotImplementedError("write your pallas kernel here")


# speedup vs baseline: 2.1970x; 2.1970x over previous
import jax
import jax.numpy as jnp
from jax.experimental import pallas as pl

SEQ_LEN = 2048
D_MODEL = 1024
BATCH = 4
BS = 256


def _kern(x_ref, t_ref, o_ref):
    t = t_ref[...]
    norm = jnp.sqrt(jnp.sum(t * t, axis=1, keepdims=True))
    scale = jnp.where(norm > 1.0, 1.0 / (norm + 1e-7), 1.0)
    emb = t * scale
    o_ref[...] = x_ref[...] + emb[:, None, :]


def kernel(x, table):
    grid = (SEQ_LEN // BS,)
    return pl.pallas_call(
        _kern,
        grid=grid,
        in_specs=[
            pl.BlockSpec((BS, BATCH, D_MODEL), lambda i: (i, 0, 0)),
            pl.BlockSpec((BS, D_MODEL), lambda i: (i, 0)),
        ],
        out_specs=pl.BlockSpec((BS, BATCH, D_MODEL), lambda i: (i, 0, 0)),
        out_shape=jax.ShapeDtypeStruct((SEQ_LEN, BATCH, D_MODEL), x.dtype),
    )(x, table)


# speedup vs baseline: 2.2527x; 1.0254x over previous
import jax
import jax.numpy as jnp
from jax.experimental import pallas as pl

SEQ_LEN = 2048
D_MODEL = 1024
BATCH = 4
BS = 512


def _kern(x_ref, t_ref, o_ref):
    t = t_ref[...]
    norm = jnp.sqrt(jnp.sum(t * t, axis=1, keepdims=True))
    scale = jnp.where(norm > 1.0, 1.0 / (norm + 1e-7), 1.0)
    emb = t * scale
    o_ref[...] = x_ref[...] + emb[:, None, :]


def kernel(x, table):
    grid = (SEQ_LEN // BS,)
    return pl.pallas_call(
        _kern,
        grid=grid,
        in_specs=[
            pl.BlockSpec((BS, BATCH, D_MODEL), lambda i: (i, 0, 0)),
            pl.BlockSpec((BS, D_MODEL), lambda i: (i, 0)),
        ],
        out_specs=pl.BlockSpec((BS, BATCH, D_MODEL), lambda i: (i, 0, 0)),
        out_shape=jax.ShapeDtypeStruct((SEQ_LEN, BATCH, D_MODEL), x.dtype),
    )(x, table)
